# baseline (speedup 1.0000x reference)
import functools

import jax
import jax.numpy as jnp
from jax import lax
from jax.experimental import pallas as pl
from jax.experimental.pallas import tpu as pltpu
from jax.experimental.pallas import tpu_sc as plsc

NC = 2
NS = 16
L = 16
NW = NC * NS

CH = 80
NCHUNK = 1250
MAXCH = 40
GG = 10
NGRP = 4
WV = 8
R = 800
DW = 144


def _mesh():
    return plsc.VectorSubcoreMesh(
        core_axis_name="c", subcore_axis_name="s",
        num_cores=NC, num_subcores=NS)


def _wid():
    return lax.axis_index("s") * NC + lax.axis_index("c")


def _n_my(wid):
    return (NCHUNK + NW - 1 - wid) // NW


def _gather_body(table_hbm, idx_hbm, out_hbm, idx_v, rows_v, sem):
    b_per_w = idx_v.shape[0]
    base = _wid() * b_per_w
    pltpu.sync_copy(idx_hbm.at[pl.ds(base, b_per_w)], idx_v)
    pltpu.async_copy(table_hbm.at[idx_v], rows_v, sem).wait()
    pltpu.sync_copy(rows_v, out_hbm.at[pl.ds(base, b_per_w)])


def _sc_gather_rows(table, idx):
    nb, d = idx.shape[0], table.shape[1]
    b_per_w = nb // NW
    return pl.kernel(
        _gather_body,
        out_type=jax.ShapeDtypeStruct((nb, d), table.dtype),
        mesh=_mesh(),
        scratch_types=[
            pltpu.VMEM((b_per_w,), jnp.int32),
            pltpu.VMEM((b_per_w, d), table.dtype),
            pltpu.SemaphoreType.DMA,
        ],
    )(table, idx)


def _expand_body(v_hbm, ids_hbm, out_hbm, idx_b, rows_b,
                 sem_i, sem_g, sem_o0, sem_o1):
    wid = _wid()
    n_my = _n_my(wid)
    h = v_hbm.shape[1]
    sem_o = (sem_o0, sem_o1)

    for j in range(MAXCH):
        @pl.when(j < n_my)
        def _():
            base = (wid + j * NW) * CH
            pltpu.async_copy(ids_hbm.at[pl.ds(base, CH)], idx_b.at[j], sem_i)
    for j in range(MAXCH):
        @pl.when(j < n_my)
        def _():
            base = (wid + j * NW) * CH
            pltpu.make_async_copy(
                ids_hbm.at[pl.ds(base, CH)], idx_b.at[j], sem_i).wait()

    def _drain_outs(grp):
        par = grp % 2
        for b in range(GG):
            j = grp * GG + b

            @pl.when(j < n_my)
            def _():
                base = (wid + j * NW) * CH
                pltpu.make_async_copy(
                    rows_b.at[par].at[b],
                    out_hbm.at[pl.ds(base, CH), pl.ds(0, h)],
                    sem_o[par]).wait()

    for grp in range(NGRP):
        par = grp % 2
        if grp >= 2:
            _drain_outs(grp - 2)
        for b in range(GG):
            j = grp * GG + b

            @pl.when(j < n_my)
            def _():
                pltpu.async_copy(v_hbm.at[idx_b.at[j]],
                                 rows_b.at[par].at[b], sem_g)
        for b in range(GG):
            j = grp * GG + b

            @pl.when(j < n_my)
            def _():
                pltpu.make_async_copy(
                    v_hbm.at[idx_b.at[j]], rows_b.at[par].at[b],
                    sem_g).wait()
        for b in range(GG):
            j = grp * GG + b

            @pl.when(j < n_my)
            def _():
                base = (wid + j * NW) * CH
                pltpu.async_copy(
                    rows_b.at[par].at[b],
                    out_hbm.at[pl.ds(base, CH), pl.ds(0, h)], sem_o[par])
    _drain_outs(NGRP - 2)
    _drain_outs(NGRP - 1)


def _sc_expand(v, ids, n):
    h = v.shape[1]
    return pl.kernel(
        _expand_body,
        out_type=jax.ShapeDtypeStruct((n, 2 * h), v.dtype),
        mesh=_mesh(),
        compiler_params=pltpu.CompilerParams(use_tc_tiling_on_sc=False),
        scratch_types=[
            pltpu.VMEM((MAXCH, CH), jnp.int32),
            pltpu.VMEM((2, GG, CH, h), v.dtype),
            pltpu.SemaphoreType.DMA,
            pltpu.SemaphoreType.DMA,
            pltpu.SemaphoreType.DMA,
            pltpu.SemaphoreType.DMA,
        ],
    )(v, ids)


def _vmat_body(g_ref, wv_ref, v_ref):
    v_ref[...] = jnp.dot(g_ref[...], wv_ref[...],
                         preferred_element_type=jnp.float32)


def _main_body(x_ref, vr_ref, wu_ref, bu_ref, we_ref, wf_ref, w_ref):
    x = x_ref[...]
    u = jnp.dot(x, wu_ref[...], preferred_element_type=jnp.float32)
    sgm = jax.nn.sigmoid(u + bu_ref[...] + vr_ref[:, 0:u.shape[1]])
    e = jnp.sum(sgm * we_ref[...], axis=1)
    w = jnp.exp(e)
    wf_ref[...] = w[:, None] * x
    w_ref[...] = w.reshape(1, 1, R)


def _segreduce_body(wf_hbm, w_hbm, ids_hbm, out_a_hbm, out_s_hbm,
                    idx_b, wf_b, w_b, tail_b, acc_a, acc_s,
                    sem_i, sem_w, sem_s):
    c = lax.axis_index("c")
    s = lax.axis_index("s")
    wid = s * NC + c
    n_my = _n_my(wid)
    nseg = acc_a.shape[0]
    d = acc_a.shape[1]
    lane = lax.iota(jnp.int32, L)
    zero = jnp.zeros((L,), jnp.float32)

    rows_per = nseg // NS
    for r in range(rows_per):
        for j in range(d // L):
            wf_b[0, r, pl.ds(j * L, L)] = zero
    for b in range(WV):
        for r in range(CH):
            tail_b[b, r] = zero
    pltpu.sync_copy(wf_b.at[0].at[pl.ds(0, rows_per)],
                    acc_a.at[pl.ds(s * rows_per, rows_per)])
    pltpu.sync_copy(tail_b.at[0].at[pl.ds(0, rows_per)],
                    acc_s.at[pl.ds(s * rows_per, rows_per)])
    plsc.subcore_barrier()

    for j in range(MAXCH):
        @pl.when(j < n_my)
        def _():
            base = (wid + j * NW) * CH
            pltpu.async_copy(ids_hbm.at[pl.ds(base, CH)], idx_b.at[j], sem_i)
    for j in range(MAXCH):
        @pl.when(j < n_my)
        def _():
            base = (wid + j * NW) * CH
            pltpu.make_async_copy(
                ids_hbm.at[pl.ds(base, CH)], idx_b.at[j], sem_i).wait()

    for wave in range(MAXCH // WV):
        for b in range(WV):
            j = wave * WV + b

            @pl.when(j < n_my)
            def _():
                base = (wid + j * NW) * CH
                pltpu.async_copy(wf_hbm.at[pl.ds(base, CH)], wf_b.at[b],
                                 sem_w)
                pltpu.async_copy(w_hbm.at[pl.ds(base, CH)], w_b.at[b],
                                 sem_w)
        for b in range(WV):
            j = wave * WV + b

            @pl.when(j < n_my)
            def _():
                base = (wid + j * NW) * CH
                pltpu.make_async_copy(
                    wf_hbm.at[pl.ds(base, CH)], wf_b.at[b], sem_w).wait()
                pltpu.make_async_copy(
                    w_hbm.at[pl.ds(base, CH)], w_b.at[b], sem_w).wait()
        for b in range(WV):
            j = wave * WV + b

            @pl.when(j < n_my)
            def _():
                for g in range(CH // L):
                    vals = w_b[b, pl.ds(g * L, L)]
                    rows = jnp.full((L,), g * L, jnp.int32) + lane
                    plsc.store_scatter(
                        tail_b,
                        [jnp.full((L,), b, jnp.int32), rows,
                         jnp.zeros((L,), jnp.int32)], vals)
        for b in range(WV):
            j = wave * WV + b

            @pl.when(j < n_my)
            def _():
                pltpu.make_async_copy(
                    wf_b.at[b], acc_a.at[idx_b.at[j]], sem_s).start(add=True)
                pltpu.make_async_copy(
                    tail_b.at[b], acc_s.at[idx_b.at[j]], sem_s).start(add=True)
        for b in range(WV):
            j = wave * WV + b

            @pl.when(j < n_my)
            def _():
                pltpu.make_async_copy(
                    wf_b.at[b], acc_a.at[idx_b.at[j]], sem_s).wait()
                pltpu.make_async_copy(
                    tail_b.at[b], acc_s.at[idx_b.at[j]], sem_s).wait()

    plsc.subcore_barrier()

    @pl.when(s == 0)
    def _dump():
        pltpu.sync_copy(acc_a, out_a_hbm.at[c])
        pltpu.sync_copy(acc_s, out_s_hbm.at[c])


def _sc_segreduce(wf, w, ids, nseg):
    n, d = wf.shape
    return pl.kernel(
        _segreduce_body,
        out_type=(jax.ShapeDtypeStruct((NC, nseg, d), jnp.float32),
                  jax.ShapeDtypeStruct((NC, nseg, L), jnp.float32)),
        mesh=_mesh(),
        compiler_params=pltpu.CompilerParams(
            needs_layout_passes=False, use_tc_tiling_on_sc=False),
        scratch_types=[
            pltpu.VMEM((MAXCH, CH), jnp.int32),
            pltpu.VMEM((WV, CH, d), jnp.float32),
            pltpu.VMEM((WV, CH), jnp.float32),
            pltpu.VMEM((WV, CH, L), jnp.float32),
            pltpu.VMEM_SHARED((nseg, d), jnp.float32),
            pltpu.VMEM_SHARED((nseg, L), jnp.float32),
            pltpu.SemaphoreType.DMA,
            pltpu.SemaphoreType.DMA,
            pltpu.SemaphoreType.DMA,
        ],
    )(wf, w, ids)


def _combine_body(pa_ref, ps_ref, out_ref):
    a = pa_ref[0] + pa_ref[1]
    sv = ps_ref[0, :, 0:1] + ps_ref[1, :, 0:1]
    out_ref[...] = jnp.where(sv > 0.0, a / sv, 0.0)


def kernel(feats, W_u, b_u, W_v, W_e, segment_ids, last_nodes):
    n, d = feats.shape
    h = W_u.shape[1]
    nseg = last_nodes.shape[0]
    ids = segment_ids.astype(jnp.int32)
    lns = last_nodes.astype(jnp.int32)

    g = _sc_gather_rows(feats, lns)

    v = pl.pallas_call(
        _vmat_body,
        out_shape=jax.ShapeDtypeStruct((nseg, h), jnp.float32),
    )(g, W_v)

    vrow = _sc_expand(v, ids, n)

    nb = n // R
    wf, w3 = pl.pallas_call(
        _main_body,
        grid=(nb,),
        in_specs=[
            pl.BlockSpec((R, d), lambda i: (i, 0)),
            pl.BlockSpec((R, 2 * h), lambda i: (i, 0)),
            pl.BlockSpec((d, h), lambda i: (0, 0)),
            pl.BlockSpec((1, h), lambda i: (0, 0)),
            pl.BlockSpec((1, h), lambda i: (0, 0)),
        ],
        out_specs=[
            pl.BlockSpec((R, d), lambda i: (i, 0)),
            pl.BlockSpec((1, 1, R), lambda i: (i, 0, 0)),
        ],
        out_shape=[
            jax.ShapeDtypeStruct((n, d), jnp.float32),
            jax.ShapeDtypeStruct((nb, 1, R), jnp.float32),
        ],
    )(feats, vrow, W_u, b_u.reshape(1, h), W_e.reshape(1, h))

    pa, ps = _sc_segreduce(wf, w3.reshape(n), ids, nseg)

    out = pl.pallas_call(
        _combine_body,
        out_shape=jax.ShapeDtypeStruct((nseg, d), jnp.float32),
    )(pa, ps)

    return out.reshape(nseg, 1, d)


# speedup vs baseline: 1.0219x; 1.0219x over previous
import functools

import jax
import jax.numpy as jnp
from jax import lax
from jax.experimental import pallas as pl
from jax.experimental.pallas import tpu as pltpu
from jax.experimental.pallas import tpu_sc as plsc

NC = 2
NS = 16
L = 16
NW = NC * NS

CH = 80
NCHUNK = 1250
MAXCH = 40
GG = 10
NGRP = 4
WV = 8
R = 1000
DW = 144


def _mesh():
    return plsc.VectorSubcoreMesh(
        core_axis_name="c", subcore_axis_name="s",
        num_cores=NC, num_subcores=NS)


def _wid():
    return lax.axis_index("s") * NC + lax.axis_index("c")


def _n_my(wid):
    return (NCHUNK + NW - 1 - wid) // NW


def _gather_body(table_hbm, idx_hbm, out_hbm, idx_v, rows_v, sem):
    b_per_w = idx_v.shape[0]
    base = _wid() * b_per_w
    pltpu.sync_copy(idx_hbm.at[pl.ds(base, b_per_w)], idx_v)
    pltpu.async_copy(table_hbm.at[idx_v], rows_v, sem).wait()
    pltpu.sync_copy(rows_v, out_hbm.at[pl.ds(base, b_per_w)])


def _sc_gather_rows(table, idx):
    nb, d = idx.shape[0], table.shape[1]
    b_per_w = nb // NW
    return pl.kernel(
        _gather_body,
        out_type=jax.ShapeDtypeStruct((nb, d), table.dtype),
        mesh=_mesh(),
        scratch_types=[
            pltpu.VMEM((b_per_w,), jnp.int32),
            pltpu.VMEM((b_per_w, d), table.dtype),
            pltpu.SemaphoreType.DMA,
        ],
    )(table, idx)


def _expand_body(v_hbm, ids_hbm, out_hbm, idx_b, rows_b,
                 sem_i, sem_g, sem_o0, sem_o1):
    wid = _wid()
    n_my = _n_my(wid)
    h = v_hbm.shape[1]
    sem_o = (sem_o0, sem_o1)

    for j in range(MAXCH):
        @pl.when(j < n_my)
        def _():
            base = (wid + j * NW) * CH
            pltpu.async_copy(ids_hbm.at[pl.ds(base, CH)], idx_b.at[j], sem_i)
    for j in range(MAXCH):
        @pl.when(j < n_my)
        def _():
            base = (wid + j * NW) * CH
            pltpu.make_async_copy(
                ids_hbm.at[pl.ds(base, CH)], idx_b.at[j], sem_i).wait()

    def _drain_outs(grp):
        par = grp % 2
        for b in range(GG):
            j = grp * GG + b

            @pl.when(j < n_my)
            def _():
                base = (wid + j * NW) * CH
                pltpu.make_async_copy(
                    rows_b.at[par].at[b],
                    out_hbm.at[pl.ds(base, CH), pl.ds(0, h)],
                    sem_o[par]).wait()

    for grp in range(NGRP):
        par = grp % 2
        if grp >= 2:
            _drain_outs(grp - 2)
        for b in range(GG):
            j = grp * GG + b

            @pl.when(j < n_my)
            def _():
                pltpu.async_copy(v_hbm.at[idx_b.at[j]],
                                 rows_b.at[par].at[b], sem_g)
        for b in range(GG):
            j = grp * GG + b

            @pl.when(j < n_my)
            def _():
                pltpu.make_async_copy(
                    v_hbm.at[idx_b.at[j]], rows_b.at[par].at[b],
                    sem_g).wait()
        for b in range(GG):
            j = grp * GG + b

            @pl.when(j < n_my)
            def _():
                base = (wid + j * NW) * CH
                pltpu.async_copy(
                    rows_b.at[par].at[b],
                    out_hbm.at[pl.ds(base, CH), pl.ds(0, h)], sem_o[par])
    _drain_outs(NGRP - 2)
    _drain_outs(NGRP - 1)


def _sc_expand(v, ids, n):
    h = v.shape[1]
    return pl.kernel(
        _expand_body,
        out_type=jax.ShapeDtypeStruct((n, 2 * h), v.dtype),
        mesh=_mesh(),
        compiler_params=pltpu.CompilerParams(use_tc_tiling_on_sc=False),
        scratch_types=[
            pltpu.VMEM((MAXCH, CH), jnp.int32),
            pltpu.VMEM((2, GG, CH, h), v.dtype),
            pltpu.SemaphoreType.DMA,
            pltpu.SemaphoreType.DMA,
            pltpu.SemaphoreType.DMA,
            pltpu.SemaphoreType.DMA,
        ],
    )(v, ids)


def _vmat_body(g_ref, wv_ref, v_ref):
    v_ref[...] = jnp.dot(g_ref[...], wv_ref[...],
                         preferred_element_type=jnp.float32)


def _main_body(x_ref, vr_ref, wu_ref, bu_ref, we_ref, wf_ref, w_ref):
    x = x_ref[...]
    u = jnp.dot(x, wu_ref[...], preferred_element_type=jnp.float32)
    sgm = jax.nn.sigmoid(u + bu_ref[...] + vr_ref[:, 0:u.shape[1]])
    e = jnp.sum(sgm * we_ref[...], axis=1)
    w = jnp.exp(e)
    wf_ref[...] = w[:, None] * x
    w_ref[...] = w.reshape(1, 1, R)


def _segreduce_body(wf_hbm, w_hbm, ids_hbm, out_a_hbm, out_s_hbm,
                    idx_b, wf_b, w_b, tail_b, acc_a, acc_s,
                    sem_i, sem_w, sem_s):
    c = lax.axis_index("c")
    s = lax.axis_index("s")
    wid = s * NC + c
    n_my = _n_my(wid)
    nseg = acc_a.shape[0]
    d = acc_a.shape[1]
    lane = lax.iota(jnp.int32, L)
    zero = jnp.zeros((L,), jnp.float32)

    rows_per = nseg // NS
    for r in range(rows_per):
        for j in range(d // L):
            wf_b[0, r, pl.ds(j * L, L)] = zero
    for b in range(WV):
        for r in range(CH):
            tail_b[b, r] = zero
    pltpu.sync_copy(wf_b.at[0].at[pl.ds(0, rows_per)],
                    acc_a.at[pl.ds(s * rows_per, rows_per)])
    pltpu.sync_copy(tail_b.at[0].at[pl.ds(0, rows_per)],
                    acc_s.at[pl.ds(s * rows_per, rows_per)])
    plsc.subcore_barrier()

    for j in range(MAXCH):
        @pl.when(j < n_my)
        def _():
            base = (wid + j * NW) * CH
            pltpu.async_copy(ids_hbm.at[pl.ds(base, CH)], idx_b.at[j], sem_i)
    for j in range(MAXCH):
        @pl.when(j < n_my)
        def _():
            base = (wid + j * NW) * CH
            pltpu.make_async_copy(
                ids_hbm.at[pl.ds(base, CH)], idx_b.at[j], sem_i).wait()

    for wave in range(MAXCH // WV):
        for b in range(WV):
            j = wave * WV + b

            @pl.when(j < n_my)
            def _():
                base = (wid + j * NW) * CH
                pltpu.async_copy(wf_hbm.at[pl.ds(base, CH)], wf_b.at[b],
                                 sem_w)
                pltpu.async_copy(w_hbm.at[pl.ds(base, CH)], w_b.at[b],
                                 sem_w)
        for b in range(WV):
            j = wave * WV + b

            @pl.when(j < n_my)
            def _():
                base = (wid + j * NW) * CH
                pltpu.make_async_copy(
                    wf_hbm.at[pl.ds(base, CH)], wf_b.at[b], sem_w).wait()
                pltpu.make_async_copy(
                    w_hbm.at[pl.ds(base, CH)], w_b.at[b], sem_w).wait()
        for b in range(WV):
            j = wave * WV + b

            @pl.when(j < n_my)
            def _():
                for g in range(CH // L):
                    vals = w_b[b, pl.ds(g * L, L)]
                    rows = jnp.full((L,), g * L, jnp.int32) + lane
                    plsc.store_scatter(
                        tail_b,
                        [jnp.full((L,), b, jnp.int32), rows,
                         jnp.zeros((L,), jnp.int32)], vals)
        for b in range(WV):
            j = wave * WV + b

            @pl.when(j < n_my)
            def _():
                pltpu.make_async_copy(
                    wf_b.at[b], acc_a.at[idx_b.at[j]], sem_s).start(add=True)
                pltpu.make_async_copy(
                    tail_b.at[b], acc_s.at[idx_b.at[j]], sem_s).start(add=True)
        for b in range(WV):
            j = wave * WV + b

            @pl.when(j < n_my)
            def _():
                pltpu.make_async_copy(
                    wf_b.at[b], acc_a.at[idx_b.at[j]], sem_s).wait()
                pltpu.make_async_copy(
                    tail_b.at[b], acc_s.at[idx_b.at[j]], sem_s).wait()

    plsc.subcore_barrier()

    @pl.when(s == 0)
    def _dump():
        pltpu.sync_copy(acc_a, out_a_hbm.at[c])
        pltpu.sync_copy(acc_s, out_s_hbm.at[c])


def _sc_segreduce(wf, w, ids, nseg):
    n, d = wf.shape
    return pl.kernel(
        _segreduce_body,
        out_type=(jax.ShapeDtypeStruct((NC, nseg, d), jnp.float32),
                  jax.ShapeDtypeStruct((NC, nseg, L), jnp.float32)),
        mesh=_mesh(),
        compiler_params=pltpu.CompilerParams(
            needs_layout_passes=False, use_tc_tiling_on_sc=False),
        scratch_types=[
            pltpu.VMEM((MAXCH, CH), jnp.int32),
            pltpu.VMEM((WV, CH, d), jnp.float32),
            pltpu.VMEM((WV, CH), jnp.float32),
            pltpu.VMEM((WV, CH, L), jnp.float32),
            pltpu.VMEM_SHARED((nseg, d), jnp.float32),
            pltpu.VMEM_SHARED((nseg, L), jnp.float32),
            pltpu.SemaphoreType.DMA,
            pltpu.SemaphoreType.DMA,
            pltpu.SemaphoreType.DMA,
        ],
    )(wf, w, ids)


def _combine_body(pa_ref, ps_ref, out_ref):
    a = pa_ref[0] + pa_ref[1]
    sv = ps_ref[0, :, 0:1] + ps_ref[1, :, 0:1]
    out_ref[...] = jnp.where(sv > 0.0, a / sv, 0.0)


def kernel(feats, W_u, b_u, W_v, W_e, segment_ids, last_nodes):
    n, d = feats.shape
    h = W_u.shape[1]
    nseg = last_nodes.shape[0]
    ids = segment_ids.astype(jnp.int32)
    lns = last_nodes.astype(jnp.int32)

    g = _sc_gather_rows(feats, lns)

    v = pl.pallas_call(
        _vmat_body,
        out_shape=jax.ShapeDtypeStruct((nseg, h), jnp.float32),
    )(g, W_v)

    vrow = _sc_expand(v, ids, n)

    nb = n // R
    wf, w3 = pl.pallas_call(
        _main_body,
        grid=(nb,),
        in_specs=[
            pl.BlockSpec((R, d), lambda i: (i, 0)),
            pl.BlockSpec((R, 2 * h), lambda i: (i, 0)),
            pl.BlockSpec((d, h), lambda i: (0, 0)),
            pl.BlockSpec((1, h), lambda i: (0, 0)),
            pl.BlockSpec((1, h), lambda i: (0, 0)),
        ],
        out_specs=[
            pl.BlockSpec((R, d), lambda i: (i, 0)),
            pl.BlockSpec((1, 1, R), lambda i: (i, 0, 0)),
        ],
        out_shape=[
            jax.ShapeDtypeStruct((n, d), jnp.float32),
            jax.ShapeDtypeStruct((nb, 1, R), jnp.float32),
        ],
    )(feats, vrow, W_u, b_u.reshape(1, h), W_e.reshape(1, h))

    pa, ps = _sc_segreduce(wf, w3.reshape(n), ids, nseg)

    out = pl.pallas_call(
        _combine_body,
        out_shape=jax.ShapeDtypeStruct((nseg, d), jnp.float32),
    )(pa, ps)

    return out.reshape(nseg, 1, d)


# speedup vs baseline: 1.0547x; 1.0321x over previous
import functools

import jax
import jax.numpy as jnp
from jax import lax
from jax.experimental import pallas as pl
from jax.experimental.pallas import tpu as pltpu
from jax.experimental.pallas import tpu_sc as plsc

NC = 2
NS = 16
L = 16
NW = NC * NS

CH = 80
NCHUNK = 1250
MAXCH = 40
GG = 10
NGRP = 4
WV = 4
R = 1000
DW = 144


def _mesh():
    return plsc.VectorSubcoreMesh(
        core_axis_name="c", subcore_axis_name="s",
        num_cores=NC, num_subcores=NS)


def _wid():
    return lax.axis_index("s") * NC + lax.axis_index("c")


def _n_my(wid):
    return (NCHUNK + NW - 1 - wid) // NW


def _gather_body(table_hbm, idx_hbm, out_hbm, idx_v, rows_v, sem):
    b_per_w = idx_v.shape[0]
    base = _wid() * b_per_w
    pltpu.sync_copy(idx_hbm.at[pl.ds(base, b_per_w)], idx_v)
    pltpu.async_copy(table_hbm.at[idx_v], rows_v, sem).wait()
    pltpu.sync_copy(rows_v, out_hbm.at[pl.ds(base, b_per_w)])


def _sc_gather_rows(table, idx):
    nb, d = idx.shape[0], table.shape[1]
    b_per_w = nb // NW
    return pl.kernel(
        _gather_body,
        out_type=jax.ShapeDtypeStruct((nb, d), table.dtype),
        mesh=_mesh(),
        scratch_types=[
            pltpu.VMEM((b_per_w,), jnp.int32),
            pltpu.VMEM((b_per_w, d), table.dtype),
            pltpu.SemaphoreType.DMA,
        ],
    )(table, idx)


def _expand_body(v_hbm, ids_hbm, out_hbm, idx_b, rows_b,
                 sem_i, sem_g, sem_o0, sem_o1):
    wid = _wid()
    n_my = _n_my(wid)
    h = v_hbm.shape[1]
    sem_o = (sem_o0, sem_o1)

    for j in range(MAXCH):
        @pl.when(j < n_my)
        def _():
            base = (wid + j * NW) * CH
            pltpu.async_copy(ids_hbm.at[pl.ds(base, CH)], idx_b.at[j], sem_i)
    for j in range(MAXCH):
        @pl.when(j < n_my)
        def _():
            base = (wid + j * NW) * CH
            pltpu.make_async_copy(
                ids_hbm.at[pl.ds(base, CH)], idx_b.at[j], sem_i).wait()

    def _drain_outs(grp):
        par = grp % 2
        for b in range(GG):
            j = grp * GG + b

            @pl.when(j < n_my)
            def _():
                base = (wid + j * NW) * CH
                pltpu.make_async_copy(
                    rows_b.at[par].at[b],
                    out_hbm.at[pl.ds(base, CH), pl.ds(0, h)],
                    sem_o[par]).wait()

    for grp in range(NGRP):
        par = grp % 2
        if grp >= 2:
            _drain_outs(grp - 2)
        for b in range(GG):
            j = grp * GG + b

            @pl.when(j < n_my)
            def _():
                pltpu.async_copy(v_hbm.at[idx_b.at[j]],
                                 rows_b.at[par].at[b], sem_g)
        for b in range(GG):
            j = grp * GG + b

            @pl.when(j < n_my)
            def _():
                pltpu.make_async_copy(
                    v_hbm.at[idx_b.at[j]], rows_b.at[par].at[b],
                    sem_g).wait()
        for b in range(GG):
            j = grp * GG + b

            @pl.when(j < n_my)
            def _():
                base = (wid + j * NW) * CH
                pltpu.async_copy(
                    rows_b.at[par].at[b],
                    out_hbm.at[pl.ds(base, CH), pl.ds(0, h)], sem_o[par])
    _drain_outs(NGRP - 2)
    _drain_outs(NGRP - 1)


def _sc_expand(v, ids, n):
    h = v.shape[1]
    return pl.kernel(
        _expand_body,
        out_type=jax.ShapeDtypeStruct((n, 2 * h), v.dtype),
        mesh=_mesh(),
        compiler_params=pltpu.CompilerParams(use_tc_tiling_on_sc=False),
        scratch_types=[
            pltpu.VMEM((MAXCH, CH), jnp.int32),
            pltpu.VMEM((2, GG, CH, h), v.dtype),
            pltpu.SemaphoreType.DMA,
            pltpu.SemaphoreType.DMA,
            pltpu.SemaphoreType.DMA,
            pltpu.SemaphoreType.DMA,
        ],
    )(v, ids)


def _vmat_body(g_ref, wv_ref, v_ref):
    v_ref[...] = jnp.dot(g_ref[...], wv_ref[...],
                         preferred_element_type=jnp.float32)


def _main_body(x_ref, vr_ref, wu_ref, bu_ref, we_ref, wf_ref, w_ref):
    x = x_ref[...]
    u = jnp.dot(x, wu_ref[...], preferred_element_type=jnp.float32)
    sgm = jax.nn.sigmoid(u + bu_ref[...] + vr_ref[:, 0:u.shape[1]])
    e = jnp.sum(sgm * we_ref[...], axis=1)
    w = jnp.exp(e)
    wf_ref[...] = w[:, None] * x
    w_ref[...] = w.reshape(1, 1, R)


def _segreduce_body(wf_hbm, w_hbm, ids_hbm, out_a_hbm, out_s_hbm,
                    idx_b, wf_b, w_b, tail_b, acc_a, acc_s,
                    sem_i, sem_w0, sem_w1, sem_s0, sem_s1):
    c = lax.axis_index("c")
    s = lax.axis_index("s")
    wid = s * NC + c
    n_my = _n_my(wid)
    nseg = acc_a.shape[0]
    d = acc_a.shape[1]
    lane = lax.iota(jnp.int32, L)
    zero = jnp.zeros((L,), jnp.float32)
    sem_w = (sem_w0, sem_w1)
    sem_s = (sem_s0, sem_s1)

    rows_per = nseg // NS
    for r in range(rows_per):
        for j in range(d // L):
            wf_b[0, 0, r, pl.ds(j * L, L)] = zero
    for par in range(2):
        for b in range(WV):
            for r in range(CH):
                tail_b[par, b, r] = zero
    pltpu.sync_copy(wf_b.at[0].at[0].at[pl.ds(0, rows_per)],
                    acc_a.at[pl.ds(s * rows_per, rows_per)])
    pltpu.sync_copy(tail_b.at[0].at[0].at[pl.ds(0, rows_per)],
                    acc_s.at[pl.ds(s * rows_per, rows_per)])
    plsc.subcore_barrier()

    for j in range(MAXCH):
        @pl.when(j < n_my)
        def _():
            base = (wid + j * NW) * CH
            pltpu.async_copy(ids_hbm.at[pl.ds(base, CH)], idx_b.at[j], sem_i)
    for j in range(MAXCH):
        @pl.when(j < n_my)
        def _():
            base = (wid + j * NW) * CH
            pltpu.make_async_copy(
                ids_hbm.at[pl.ds(base, CH)], idx_b.at[j], sem_i).wait()

    def _drain_scatters(wave):
        par = wave % 2
        for b in range(WV):
            j = wave * WV + b

            @pl.when(j < n_my)
            def _():
                pltpu.make_async_copy(
                    wf_b.at[par].at[b], acc_a.at[idx_b.at[j]],
                    sem_s[par]).wait()
                pltpu.make_async_copy(
                    tail_b.at[par].at[b], acc_s.at[idx_b.at[j]],
                    sem_s[par]).wait()

    n_waves = (MAXCH + WV - 1) // WV
    for wave in range(n_waves):
        par = wave % 2
        if wave >= 2:
            _drain_scatters(wave - 2)
        for b in range(WV):
            j = wave * WV + b

            @pl.when(j < n_my)
            def _():
                base = (wid + j * NW) * CH
                pltpu.async_copy(wf_hbm.at[pl.ds(base, CH)],
                                 wf_b.at[par].at[b], sem_w[par])
                pltpu.async_copy(w_hbm.at[pl.ds(base, CH)],
                                 w_b.at[par].at[b], sem_w[par])
        for b in range(WV):
            j = wave * WV + b

            @pl.when(j < n_my)
            def _():
                base = (wid + j * NW) * CH
                pltpu.make_async_copy(
                    wf_hbm.at[pl.ds(base, CH)], wf_b.at[par].at[b],
                    sem_w[par]).wait()
                pltpu.make_async_copy(
                    w_hbm.at[pl.ds(base, CH)], w_b.at[par].at[b],
                    sem_w[par]).wait()
        for b in range(WV):
            j = wave * WV + b

            @pl.when(j < n_my)
            def _():
                for g in range(CH // L):
                    vals = w_b[par, b, pl.ds(g * L, L)]
                    rows = jnp.full((L,), g * L, jnp.int32) + lane
                    plsc.store_scatter(
                        tail_b,
                        [jnp.full((L,), par, jnp.int32),
                         jnp.full((L,), b, jnp.int32), rows,
                         jnp.zeros((L,), jnp.int32)], vals)
        for b in range(WV):
            j = wave * WV + b

            @pl.when(j < n_my)
            def _():
                pltpu.make_async_copy(
                    wf_b.at[par].at[b], acc_a.at[idx_b.at[j]],
                    sem_s[par]).start(add=True)
                pltpu.make_async_copy(
                    tail_b.at[par].at[b], acc_s.at[idx_b.at[j]],
                    sem_s[par]).start(add=True)
    if n_waves >= 2:
        _drain_scatters(n_waves - 2)
    _drain_scatters(n_waves - 1)

    plsc.subcore_barrier()

    @pl.when(s == 0)
    def _dump():
        pltpu.sync_copy(acc_a, out_a_hbm.at[c])
        pltpu.sync_copy(acc_s, out_s_hbm.at[c])


def _sc_segreduce(wf, w, ids, nseg):
    n, d = wf.shape
    return pl.kernel(
        _segreduce_body,
        out_type=(jax.ShapeDtypeStruct((NC, nseg, d), jnp.float32),
                  jax.ShapeDtypeStruct((NC, nseg, L), jnp.float32)),
        mesh=_mesh(),
        compiler_params=pltpu.CompilerParams(
            needs_layout_passes=False, use_tc_tiling_on_sc=False),
        scratch_types=[
            pltpu.VMEM((MAXCH, CH), jnp.int32),
            pltpu.VMEM((2, WV, CH, d), jnp.float32),
            pltpu.VMEM((2, WV, CH), jnp.float32),
            pltpu.VMEM((2, WV, CH, L), jnp.float32),
            pltpu.VMEM_SHARED((nseg, d), jnp.float32),
            pltpu.VMEM_SHARED((nseg, L), jnp.float32),
            pltpu.SemaphoreType.DMA,
            pltpu.SemaphoreType.DMA,
            pltpu.SemaphoreType.DMA,
            pltpu.SemaphoreType.DMA,
            pltpu.SemaphoreType.DMA,
        ],
    )(wf, w, ids)


def _combine_body(pa_ref, ps_ref, out_ref):
    a = pa_ref[0] + pa_ref[1]
    sv = ps_ref[0, :, 0:1] + ps_ref[1, :, 0:1]
    out_ref[...] = jnp.where(sv > 0.0, a / sv, 0.0)


def kernel(feats, W_u, b_u, W_v, W_e, segment_ids, last_nodes):
    n, d = feats.shape
    h = W_u.shape[1]
    nseg = last_nodes.shape[0]
    ids = segment_ids.astype(jnp.int32)
    lns = last_nodes.astype(jnp.int32)

    g = _sc_gather_rows(feats, lns)

    v = pl.pallas_call(
        _vmat_body,
        out_shape=jax.ShapeDtypeStruct((nseg, h), jnp.float32),
    )(g, W_v)

    vrow = _sc_expand(v, ids, n)

    nb = n // R
    wf, w3 = pl.pallas_call(
        _main_body,
        grid=(nb,),
        in_specs=[
            pl.BlockSpec((R, d), lambda i: (i, 0)),
            pl.BlockSpec((R, 2 * h), lambda i: (i, 0)),
            pl.BlockSpec((d, h), lambda i: (0, 0)),
            pl.BlockSpec((1, h), lambda i: (0, 0)),
            pl.BlockSpec((1, h), lambda i: (0, 0)),
        ],
        out_specs=[
            pl.BlockSpec((R, d), lambda i: (i, 0)),
            pl.BlockSpec((1, 1, R), lambda i: (i, 0, 0)),
        ],
        out_shape=[
            jax.ShapeDtypeStruct((n, d), jnp.float32),
            jax.ShapeDtypeStruct((nb, 1, R), jnp.float32),
        ],
    )(feats, vrow, W_u, b_u.reshape(1, h), W_e.reshape(1, h))

    pa, ps = _sc_segreduce(wf, w3.reshape(n), ids, nseg)

    out = pl.pallas_call(
        _combine_body,
        out_shape=jax.ShapeDtypeStruct((nseg, d), jnp.float32),
    )(pa, ps)

    return out.reshape(nseg, 1, d)


# speedup vs baseline: 1.1363x; 1.0774x over previous
import functools

import jax
import jax.numpy as jnp
from jax import lax
from jax.experimental import pallas as pl
from jax.experimental.pallas import tpu as pltpu
from jax.experimental.pallas import tpu_sc as plsc

NC = 2
NS = 16
L = 16
NW = NC * NS

CH = 80
NCHUNK = 1250
MAXCH = 40
GG = 10
NGRP = 4
WV = 4
R = 1000
DW = 144


def _mesh():
    return plsc.VectorSubcoreMesh(
        core_axis_name="c", subcore_axis_name="s",
        num_cores=NC, num_subcores=NS)


def _wid():
    return lax.axis_index("s") * NC + lax.axis_index("c")


def _n_my(wid, nch):
    return (nch + NW - 1 - wid) // NW


def _gather_body(table_hbm, idx_hbm, out_hbm, idx_v, rows_v, sem):
    b_per_w = idx_v.shape[0]
    base = _wid() * b_per_w
    pltpu.sync_copy(idx_hbm.at[pl.ds(base, b_per_w)], idx_v)
    pltpu.async_copy(table_hbm.at[idx_v], rows_v, sem).wait()
    pltpu.sync_copy(rows_v, out_hbm.at[pl.ds(base, b_per_w)])


def _sc_gather_rows(table, idx):
    nb, d = idx.shape[0], table.shape[1]
    b_per_w = nb // NW
    return pl.kernel(
        _gather_body,
        out_type=jax.ShapeDtypeStruct((nb, d), table.dtype),
        mesh=_mesh(),
        scratch_types=[
            pltpu.VMEM((b_per_w,), jnp.int32),
            pltpu.VMEM((b_per_w, d), table.dtype),
            pltpu.SemaphoreType.DMA,
        ],
    )(table, idx)


def _expand_body(v_hbm, ids_hbm, out_hbm, idx_b, rows_b,
                 sem_i, sem_g, sem_o0, sem_o1, *, row0):
    wid = _wid()
    nch = out_hbm.shape[0] // CH
    maxch = (nch + NW - 1) // NW
    ngrp = (maxch + GG - 1) // GG
    n_my = _n_my(wid, nch)
    h = v_hbm.shape[1]
    sem_o = (sem_o0, sem_o1)

    for j in range(maxch):
        @pl.when(j < n_my)
        def _():
            base = (wid + j * NW) * CH
            pltpu.async_copy(ids_hbm.at[pl.ds(row0 + base, CH)],
                             idx_b.at[j], sem_i)
    for j in range(maxch):
        @pl.when(j < n_my)
        def _():
            base = (wid + j * NW) * CH
            pltpu.make_async_copy(
                ids_hbm.at[pl.ds(row0 + base, CH)], idx_b.at[j],
                sem_i).wait()

    def _drain_outs(grp):
        par = grp % 2
        for b in range(GG):
            j = grp * GG + b

            @pl.when(j < n_my)
            def _():
                base = (wid + j * NW) * CH
                pltpu.make_async_copy(
                    rows_b.at[par].at[b],
                    out_hbm.at[pl.ds(base, CH), pl.ds(0, h)],
                    sem_o[par]).wait()

    for grp in range(ngrp):
        par = grp % 2
        if grp >= 2:
            _drain_outs(grp - 2)
        for b in range(GG):
            j = grp * GG + b

            @pl.when(j < n_my)
            def _():
                pltpu.async_copy(v_hbm.at[idx_b.at[j]],
                                 rows_b.at[par].at[b], sem_g)
        for b in range(GG):
            j = grp * GG + b

            @pl.when(j < n_my)
            def _():
                pltpu.make_async_copy(
                    v_hbm.at[idx_b.at[j]], rows_b.at[par].at[b],
                    sem_g).wait()
        for b in range(GG):
            j = grp * GG + b

            @pl.when(j < n_my)
            def _():
                base = (wid + j * NW) * CH
                pltpu.async_copy(
                    rows_b.at[par].at[b],
                    out_hbm.at[pl.ds(base, CH), pl.ds(0, h)], sem_o[par])
    if ngrp >= 2:
        _drain_outs(ngrp - 2)
    _drain_outs(ngrp - 1)


def _sc_expand(v, ids, n, row0=0):
    h = v.shape[1]
    return pl.kernel(
        functools.partial(_expand_body, row0=row0),
        out_type=jax.ShapeDtypeStruct((n, 2 * h), v.dtype),
        mesh=_mesh(),
        compiler_params=pltpu.CompilerParams(use_tc_tiling_on_sc=False),
        scratch_types=[
            pltpu.VMEM((MAXCH, CH), jnp.int32),
            pltpu.VMEM((2, GG, CH, h), v.dtype),
            pltpu.SemaphoreType.DMA,
            pltpu.SemaphoreType.DMA,
            pltpu.SemaphoreType.DMA,
            pltpu.SemaphoreType.DMA,
        ],
    )(v, ids)


def _vmat_body(g_ref, wv_ref, v_ref):
    v_ref[...] = jnp.dot(g_ref[...], wv_ref[...],
                         preferred_element_type=jnp.float32)


def _main_body(x_ref, vr_ref, wu_ref, bu_ref, we_ref, wf_ref, w_ref):
    x = x_ref[...]
    u = jnp.dot(x, wu_ref[...], preferred_element_type=jnp.float32)
    sgm = jax.nn.sigmoid(u + bu_ref[...] + vr_ref[:, 0:u.shape[1]])
    e = jnp.sum(sgm * we_ref[...], axis=1)
    w = jnp.exp(e)
    wf_ref[...] = w[:, None] * x
    w_ref[...] = w.reshape(1, 1, R)


def _segreduce_body(wf_hbm, w_hbm, ids_hbm, out_a_hbm, out_s_hbm,
                    idx_b, wf_b, w_b, tail_b, acc_a, acc_s,
                    sem_i, sem_w0, sem_w1, sem_s0, sem_s1, *, row0):
    c = lax.axis_index("c")
    s = lax.axis_index("s")
    wid = s * NC + c
    nch = wf_hbm.shape[0] // CH
    maxch = (nch + NW - 1) // NW
    n_my = _n_my(wid, nch)
    nseg = acc_a.shape[0]
    d = acc_a.shape[1]
    lane = lax.iota(jnp.int32, L)
    zero = jnp.zeros((L,), jnp.float32)
    sem_w = (sem_w0, sem_w1)
    sem_s = (sem_s0, sem_s1)

    rows_per = nseg // NS
    for r in range(rows_per):
        for j in range(d // L):
            wf_b[0, 0, r, pl.ds(j * L, L)] = zero
    for par in range(2):
        for b in range(WV):
            for r in range(CH):
                tail_b[par, b, r] = zero
    pltpu.sync_copy(wf_b.at[0].at[0].at[pl.ds(0, rows_per)],
                    acc_a.at[pl.ds(s * rows_per, rows_per)])
    pltpu.sync_copy(tail_b.at[0].at[0].at[pl.ds(0, rows_per)],
                    acc_s.at[pl.ds(s * rows_per, rows_per)])
    plsc.subcore_barrier()

    for j in range(maxch):
        @pl.when(j < n_my)
        def _():
            base = (wid + j * NW) * CH
            pltpu.async_copy(ids_hbm.at[pl.ds(row0 + base, CH)],
                             idx_b.at[j], sem_i)
    for j in range(maxch):
        @pl.when(j < n_my)
        def _():
            base = (wid + j * NW) * CH
            pltpu.make_async_copy(
                ids_hbm.at[pl.ds(row0 + base, CH)], idx_b.at[j],
                sem_i).wait()

    def _drain_scatters(wave):
        par = wave % 2
        for b in range(WV):
            j = wave * WV + b

            @pl.when(j < n_my)
            def _():
                pltpu.make_async_copy(
                    wf_b.at[par].at[b], acc_a.at[idx_b.at[j]],
                    sem_s[par]).wait()
                pltpu.make_async_copy(
                    tail_b.at[par].at[b], acc_s.at[idx_b.at[j]],
                    sem_s[par]).wait()

    n_waves = (maxch + WV - 1) // WV
    for wave in range(n_waves):
        par = wave % 2
        if wave >= 2:
            _drain_scatters(wave - 2)
        for b in range(WV):
            j = wave * WV + b

            @pl.when(j < n_my)
            def _():
                base = (wid + j * NW) * CH
                pltpu.async_copy(wf_hbm.at[pl.ds(base, CH)],
                                 wf_b.at[par].at[b], sem_w[par])
                pltpu.async_copy(w_hbm.at[pl.ds(base, CH)],
                                 w_b.at[par].at[b], sem_w[par])
        for b in range(WV):
            j = wave * WV + b

            @pl.when(j < n_my)
            def _():
                base = (wid + j * NW) * CH
                pltpu.make_async_copy(
                    wf_hbm.at[pl.ds(base, CH)], wf_b.at[par].at[b],
                    sem_w[par]).wait()
                pltpu.make_async_copy(
                    w_hbm.at[pl.ds(base, CH)], w_b.at[par].at[b],
                    sem_w[par]).wait()
        for b in range(WV):
            j = wave * WV + b

            @pl.when(j < n_my)
            def _():
                for g in range(CH // L):
                    vals = w_b[par, b, pl.ds(g * L, L)]
                    rows = jnp.full((L,), g * L, jnp.int32) + lane
                    plsc.store_scatter(
                        tail_b,
                        [jnp.full((L,), par, jnp.int32),
                         jnp.full((L,), b, jnp.int32), rows,
                         jnp.zeros((L,), jnp.int32)], vals)
        for b in range(WV):
            j = wave * WV + b

            @pl.when(j < n_my)
            def _():
                pltpu.make_async_copy(
                    wf_b.at[par].at[b], acc_a.at[idx_b.at[j]],
                    sem_s[par]).start(add=True)
                pltpu.make_async_copy(
                    tail_b.at[par].at[b], acc_s.at[idx_b.at[j]],
                    sem_s[par]).start(add=True)
    if n_waves >= 2:
        _drain_scatters(n_waves - 2)
    _drain_scatters(n_waves - 1)

    plsc.subcore_barrier()

    @pl.when(s == 0)
    def _dump():
        pltpu.sync_copy(acc_a, out_a_hbm.at[c])
        pltpu.sync_copy(acc_s, out_s_hbm.at[c])


def _sc_segreduce(wf, w, ids, nseg, row0=0):
    n, d = wf.shape
    return pl.kernel(
        functools.partial(_segreduce_body, row0=row0),
        out_type=(jax.ShapeDtypeStruct((NC, nseg, d), jnp.float32),
                  jax.ShapeDtypeStruct((NC, nseg, L), jnp.float32)),
        mesh=_mesh(),
        compiler_params=pltpu.CompilerParams(
            needs_layout_passes=False, use_tc_tiling_on_sc=False),
        scratch_types=[
            pltpu.VMEM((MAXCH, CH), jnp.int32),
            pltpu.VMEM((2, WV, CH, d), jnp.float32),
            pltpu.VMEM((2, WV, CH), jnp.float32),
            pltpu.VMEM((2, WV, CH, L), jnp.float32),
            pltpu.VMEM_SHARED((nseg, d), jnp.float32),
            pltpu.VMEM_SHARED((nseg, L), jnp.float32),
            pltpu.SemaphoreType.DMA,
            pltpu.SemaphoreType.DMA,
            pltpu.SemaphoreType.DMA,
            pltpu.SemaphoreType.DMA,
            pltpu.SemaphoreType.DMA,
        ],
    )(wf, w, ids)


def _combine_body(pa0_ref, ps0_ref, pa1_ref, ps1_ref, out_ref):
    a = (pa0_ref[0] + pa0_ref[1]) + (pa1_ref[0] + pa1_ref[1])
    sv = (ps0_ref[0, :, 0:1] + ps0_ref[1, :, 0:1]
          + ps1_ref[0, :, 0:1] + ps1_ref[1, :, 0:1])
    out_ref[...] = jnp.where(sv > 0.0, a / sv, 0.0)


def kernel(feats, W_u, b_u, W_v, W_e, segment_ids, last_nodes):
    n, d = feats.shape
    h = W_u.shape[1]
    nseg = last_nodes.shape[0]
    ids = segment_ids.astype(jnp.int32)
    lns = last_nodes.astype(jnp.int32)

    g = _sc_gather_rows(feats, lns)

    v = pl.pallas_call(
        _vmat_body,
        out_shape=jax.ShapeDtypeStruct((nseg, h), jnp.float32),
    )(g, W_v)

    n2 = n // 2
    nb2 = n2 // R
    halves = []
    vrows = [_sc_expand(v, ids, n2, row0=hh * n2) for hh in range(2)]
    for hh in range(2):
        off = hh * nb2
        wf, w3 = pl.pallas_call(
            _main_body,
            grid=(nb2,),
            in_specs=[
                pl.BlockSpec((R, d), lambda i, o=off: (i + o, 0)),
                pl.BlockSpec((R, 2 * h), lambda i: (i, 0)),
                pl.BlockSpec((d, h), lambda i: (0, 0)),
                pl.BlockSpec((1, h), lambda i: (0, 0)),
                pl.BlockSpec((1, h), lambda i: (0, 0)),
            ],
            out_specs=[
                pl.BlockSpec((R, d), lambda i: (i, 0)),
                pl.BlockSpec((1, 1, R), lambda i: (i, 0, 0)),
            ],
            out_shape=[
                jax.ShapeDtypeStruct((n2, d), jnp.float32),
                jax.ShapeDtypeStruct((nb2, 1, R), jnp.float32),
            ],
        )(feats, vrows[hh], W_u, b_u.reshape(1, h), W_e.reshape(1, h))
        halves.append(
            _sc_segreduce(wf, w3.reshape(n2), ids, nseg, row0=hh * n2))

    out = pl.pallas_call(
        _combine_body,
        out_shape=jax.ShapeDtypeStruct((nseg, d), jnp.float32),
    )(halves[0][0], halves[0][1], halves[1][0], halves[1][1])

    return out.reshape(nseg, 1, d)


# speedup vs baseline: 1.1406x; 1.0038x over previous
import functools

import jax
import jax.numpy as jnp
from jax import lax
from jax.experimental import pallas as pl
from jax.experimental.pallas import tpu as pltpu
from jax.experimental.pallas import tpu_sc as plsc

NC = 2
NS = 16
L = 16
NW = NC * NS

CH = 80
MAXCH = 40
GG = 10
WV = 4
R = 1000
DW = 144


def _mesh():
    return plsc.VectorSubcoreMesh(
        core_axis_name="c", subcore_axis_name="s",
        num_cores=NC, num_subcores=NS)


def _wid():
    return lax.axis_index("s") * NC + lax.axis_index("c")


def _n_my(wid, nch):
    return (nch + NW - 1 - wid) // NW


def _gather_body(table_hbm, idx_hbm, out_hbm, idx_v, rows_v, sem):
    b_per_w = idx_v.shape[0]
    base = _wid() * b_per_w
    pltpu.sync_copy(idx_hbm.at[pl.ds(base, b_per_w)], idx_v)
    pltpu.async_copy(table_hbm.at[idx_v], rows_v, sem).wait()
    pltpu.sync_copy(rows_v, out_hbm.at[pl.ds(base, b_per_w)])


def _sc_gather_rows(table, idx):
    nb, d = idx.shape[0], table.shape[1]
    b_per_w = nb // NW
    return pl.kernel(
        _gather_body,
        out_type=jax.ShapeDtypeStruct((nb, d), table.dtype),
        mesh=_mesh(),
        scratch_types=[
            pltpu.VMEM((b_per_w,), jnp.int32),
            pltpu.VMEM((b_per_w, d), table.dtype),
            pltpu.SemaphoreType.DMA,
        ],
    )(table, idx)


def _expand_body(v_hbm, ids_hbm, out_hbm, idx_b, rows_b,
                 sem_i, sem_g, sem_o0, sem_o1, *, row0):
    wid = _wid()
    nch = out_hbm.shape[0] // CH
    maxch = (nch + NW - 1) // NW
    ngrp = (maxch + GG - 1) // GG
    n_my = _n_my(wid, nch)
    h = v_hbm.shape[1]
    sem_o = (sem_o0, sem_o1)

    for j in range(maxch):
        @pl.when(j < n_my)
        def _():
            base = (wid + j * NW) * CH
            pltpu.async_copy(ids_hbm.at[pl.ds(row0 + base, CH)],
                             idx_b.at[j], sem_i)
    for j in range(maxch):
        @pl.when(j < n_my)
        def _():
            base = (wid + j * NW) * CH
            pltpu.make_async_copy(
                ids_hbm.at[pl.ds(row0 + base, CH)], idx_b.at[j],
                sem_i).wait()

    def _drain_outs(grp):
        par = grp % 2
        for b in range(GG):
            j = grp * GG + b

            @pl.when(j < n_my)
            def _():
                base = (wid + j * NW) * CH
                pltpu.make_async_copy(
                    rows_b.at[par].at[b],
                    out_hbm.at[pl.ds(base, CH), pl.ds(0, h)],
                    sem_o[par]).wait()

    for grp in range(ngrp):
        par = grp % 2
        if grp >= 2:
            _drain_outs(grp - 2)
        for b in range(GG):
            j = grp * GG + b

            @pl.when(j < n_my)
            def _():
                pltpu.async_copy(v_hbm.at[idx_b.at[j]],
                                 rows_b.at[par].at[b], sem_g)
        for b in range(GG):
            j = grp * GG + b

            @pl.when(j < n_my)
            def _():
                pltpu.make_async_copy(
                    v_hbm.at[idx_b.at[j]], rows_b.at[par].at[b],
                    sem_g).wait()
        for b in range(GG):
            j = grp * GG + b

            @pl.when(j < n_my)
            def _():
                base = (wid + j * NW) * CH
                pltpu.async_copy(
                    rows_b.at[par].at[b],
                    out_hbm.at[pl.ds(base, CH), pl.ds(0, h)], sem_o[par])
    if ngrp >= 2:
        _drain_outs(ngrp - 2)
    _drain_outs(ngrp - 1)


def _sc_expand(v, ids, n, row0=0):
    h = v.shape[1]
    return pl.kernel(
        functools.partial(_expand_body, row0=row0),
        out_type=jax.ShapeDtypeStruct((n, 2 * h), v.dtype),
        mesh=_mesh(),
        compiler_params=pltpu.CompilerParams(use_tc_tiling_on_sc=False),
        scratch_types=[
            pltpu.VMEM((MAXCH, CH), jnp.int32),
            pltpu.VMEM((2, GG, CH, h), v.dtype),
            pltpu.SemaphoreType.DMA,
            pltpu.SemaphoreType.DMA,
            pltpu.SemaphoreType.DMA,
            pltpu.SemaphoreType.DMA,
        ],
    )(v, ids)


def _vmat_body(g_ref, wv_ref, v_ref):
    v_ref[...] = jnp.dot(g_ref[...], wv_ref[...],
                         preferred_element_type=jnp.float32)


def _main_body(x_ref, vr_ref, wu_ref, bu_ref, we_ref, wf_ref, w_ref):
    x = x_ref[...]
    u = jnp.dot(x, wu_ref[...], preferred_element_type=jnp.float32)
    sgm = jax.nn.sigmoid(u + bu_ref[...] + vr_ref[:, 0:u.shape[1]])
    e = jnp.sum(sgm * we_ref[...], axis=1)
    w = jnp.exp(e)
    wf_ref[...] = w[:, None] * x
    w_ref[...] = w.reshape(1, 1, R)


def _segreduce_body(wf_hbm, w_hbm, ids_hbm, out_a_hbm, out_s_hbm,
                    idx_b, wf_b, w_b, tail_b, acc_a, acc_s,
                    sem_i, sem_w0, sem_w1, sem_s0, sem_s1, *, row0):
    c = lax.axis_index("c")
    s = lax.axis_index("s")
    wid = s * NC + c
    nch = wf_hbm.shape[0] // CH
    maxch = (nch + NW - 1) // NW
    n_my = _n_my(wid, nch)
    nseg = acc_a.shape[0]
    d = acc_a.shape[1]
    lane = lax.iota(jnp.int32, L)
    zero = jnp.zeros((L,), jnp.float32)
    sem_w = (sem_w0, sem_w1)
    sem_s = (sem_s0, sem_s1)

    rows_per = nseg // NS
    for r in range(rows_per):
        for j in range(d // L):
            wf_b[0, 0, r, pl.ds(j * L, L)] = zero
    for par in range(2):
        for b in range(WV):
            for r in range(CH):
                tail_b[par, b, r] = zero
    pltpu.sync_copy(wf_b.at[0].at[0].at[pl.ds(0, rows_per)],
                    acc_a.at[pl.ds(s * rows_per, rows_per)])
    pltpu.sync_copy(tail_b.at[0].at[0].at[pl.ds(0, rows_per)],
                    acc_s.at[pl.ds(s * rows_per, rows_per)])
    plsc.subcore_barrier()

    for j in range(maxch):
        @pl.when(j < n_my)
        def _():
            base = (wid + j * NW) * CH
            pltpu.async_copy(ids_hbm.at[pl.ds(row0 + base, CH)],
                             idx_b.at[j], sem_i)
    for j in range(maxch):
        @pl.when(j < n_my)
        def _():
            base = (wid + j * NW) * CH
            pltpu.make_async_copy(
                ids_hbm.at[pl.ds(row0 + base, CH)], idx_b.at[j],
                sem_i).wait()

    def _drain_scatters(wave):
        par = wave % 2
        for b in range(WV):
            j = wave * WV + b

            @pl.when(j < n_my)
            def _():
                pltpu.make_async_copy(
                    wf_b.at[par].at[b], acc_a.at[idx_b.at[j]],
                    sem_s[par]).wait()
                pltpu.make_async_copy(
                    tail_b.at[par].at[b], acc_s.at[idx_b.at[j]],
                    sem_s[par]).wait()

    n_waves = (maxch + WV - 1) // WV
    for wave in range(n_waves):
        par = wave % 2
        if wave >= 2:
            _drain_scatters(wave - 2)
        for b in range(WV):
            j = wave * WV + b

            @pl.when(j < n_my)
            def _():
                base = (wid + j * NW) * CH
                pltpu.async_copy(wf_hbm.at[pl.ds(base, CH)],
                                 wf_b.at[par].at[b], sem_w[par])
                pltpu.async_copy(w_hbm.at[pl.ds(base, CH)],
                                 w_b.at[par].at[b], sem_w[par])
        for b in range(WV):
            j = wave * WV + b

            @pl.when(j < n_my)
            def _():
                base = (wid + j * NW) * CH
                pltpu.make_async_copy(
                    wf_hbm.at[pl.ds(base, CH)], wf_b.at[par].at[b],
                    sem_w[par]).wait()
                pltpu.make_async_copy(
                    w_hbm.at[pl.ds(base, CH)], w_b.at[par].at[b],
                    sem_w[par]).wait()
        for b in range(WV):
            j = wave * WV + b

            @pl.when(j < n_my)
            def _():
                for g in range(CH // L):
                    vals = w_b[par, b, pl.ds(g * L, L)]
                    rows = jnp.full((L,), g * L, jnp.int32) + lane
                    plsc.store_scatter(
                        tail_b,
                        [jnp.full((L,), par, jnp.int32),
                         jnp.full((L,), b, jnp.int32), rows,
                         jnp.zeros((L,), jnp.int32)], vals)
        for b in range(WV):
            j = wave * WV + b

            @pl.when(j < n_my)
            def _():
                pltpu.make_async_copy(
                    wf_b.at[par].at[b], acc_a.at[idx_b.at[j]],
                    sem_s[par]).start(add=True)
                pltpu.make_async_copy(
                    tail_b.at[par].at[b], acc_s.at[idx_b.at[j]],
                    sem_s[par]).start(add=True)
    if n_waves >= 2:
        _drain_scatters(n_waves - 2)
    _drain_scatters(n_waves - 1)

    plsc.subcore_barrier()

    @pl.when(s == 0)
    def _dump():
        pltpu.sync_copy(acc_a, out_a_hbm.at[c])
        pltpu.sync_copy(acc_s, out_s_hbm.at[c])


def _sc_segreduce(wf, w, ids, nseg, row0=0):
    n, d = wf.shape
    return pl.kernel(
        functools.partial(_segreduce_body, row0=row0),
        out_type=(jax.ShapeDtypeStruct((NC, nseg, d), jnp.float32),
                  jax.ShapeDtypeStruct((NC, nseg, L), jnp.float32)),
        mesh=_mesh(),
        compiler_params=pltpu.CompilerParams(
            needs_layout_passes=False, use_tc_tiling_on_sc=False),
        scratch_types=[
            pltpu.VMEM((MAXCH, CH), jnp.int32),
            pltpu.VMEM((2, WV, CH, d), jnp.float32),
            pltpu.VMEM((2, WV, CH), jnp.float32),
            pltpu.VMEM((2, WV, CH, L), jnp.float32),
            pltpu.VMEM_SHARED((nseg, d), jnp.float32),
            pltpu.VMEM_SHARED((nseg, L), jnp.float32),
            pltpu.SemaphoreType.DMA,
            pltpu.SemaphoreType.DMA,
            pltpu.SemaphoreType.DMA,
            pltpu.SemaphoreType.DMA,
            pltpu.SemaphoreType.DMA,
        ],
    )(wf, w, ids)


def _combine_body(pa0_ref, ps0_ref, pa1_ref, ps1_ref, out_ref):
    a = (pa0_ref[0] + pa0_ref[1]) + (pa1_ref[0] + pa1_ref[1])
    sv = (ps0_ref[0, :, 0:1] + ps0_ref[1, :, 0:1]
          + ps1_ref[0, :, 0:1] + ps1_ref[1, :, 0:1])
    out_ref[...] = jnp.where(sv > 0.0, a / sv, 0.0)


def kernel(feats, W_u, b_u, W_v, W_e, segment_ids, last_nodes):
    n, d = feats.shape
    h = W_u.shape[1]
    nseg = last_nodes.shape[0]
    ids = segment_ids.astype(jnp.int32)
    lns = last_nodes.astype(jnp.int32)

    g = _sc_gather_rows(feats, lns)

    v = pl.pallas_call(
        _vmat_body,
        out_shape=jax.ShapeDtypeStruct((nseg, h), jnp.float32),
    )(g, W_v)

    n2 = n // 2
    nb2 = n2 // R
    halves = []
    vrows = [_sc_expand(v, ids, n2, row0=hh * n2) for hh in range(2)]
    for hh in range(2):
        off = hh * nb2
        wf, w3 = pl.pallas_call(
            _main_body,
            grid=(nb2,),
            in_specs=[
                pl.BlockSpec((R, d), lambda i, o=off: (i + o, 0)),
                pl.BlockSpec((R, 2 * h), lambda i: (i, 0)),
                pl.BlockSpec((d, h), lambda i: (0, 0)),
                pl.BlockSpec((1, h), lambda i: (0, 0)),
                pl.BlockSpec((1, h), lambda i: (0, 0)),
            ],
            out_specs=[
                pl.BlockSpec((R, d), lambda i: (i, 0)),
                pl.BlockSpec((1, 1, R), lambda i: (i, 0, 0)),
            ],
            out_shape=[
                jax.ShapeDtypeStruct((n2, d), jnp.float32),
                jax.ShapeDtypeStruct((nb2, 1, R), jnp.float32),
            ],
        )(feats, vrows[hh], W_u, b_u.reshape(1, h), W_e.reshape(1, h))
        halves.append(
            _sc_segreduce(wf, w3.reshape(n2), ids, nseg, row0=hh * n2))

    out = pl.pallas_call(
        _combine_body,
        out_shape=jax.ShapeDtypeStruct((nseg, d), jnp.float32),
    )(halves[0][0], halves[0][1], halves[1][0], halves[1][1])

    return out.reshape(nseg, 1, d)
